# baseline (speedup 1.0000x reference)
import functools

import jax
import jax.numpy as jnp
from jax import lax
from jax.experimental import pallas as pl
from jax.experimental.pallas import tpu as pltpu
from jax.experimental.pallas import tpu_sc as plsc

NC = 2
NS = 16
NW = NC * NS
LANES = 16

CH = 80
C_FEAT = 128


def _mesh():
    return plsc.VectorSubcoreMesh(
        core_axis_name="c", subcore_axis_name="s",
        num_cores=NC, num_subcores=NS)


def _make_kin_kernel(E, N, M, NP, EPTP):
    EPT = E // NW
    NITER = EPT // LANES

    @functools.partial(
        pl.kernel, mesh=_mesh(),
        compiler_params=pltpu.CompilerParams(needs_layout_passes=False),
        out_type=(jax.ShapeDtypeStruct((4, NW * EPTP), jnp.float32),
                  jax.ShapeDtypeStruct((NW, NP), jnp.int32)),
        scratch_types=[
            pltpu.VMEM((EPT,), jnp.int32),
            pltpu.VMEM((EPT,), jnp.int32),
            pltpu.VMEM((N * 2,), jnp.float32),
            pltpu.VMEM((M * 2,), jnp.float32),
            pltpu.VMEM((4, EPTP), jnp.float32),
            pltpu.VMEM((NP,), jnp.int32),
        ],
    )
    def kin_kernel(dst_hbm, src_hbm, phys_hbm, lat_hbm, kin_hbm, cnt_hbm,
                   dstv, srcv, physv, latv, kinv, hist):
        c = lax.axis_index("c")
        s = lax.axis_index("s")
        base = (s * NC + c) * EPT
        pltpu.sync_copy(dst_hbm.at[pl.ds(base, EPT)], dstv)
        pltpu.sync_copy(src_hbm.at[pl.ds(base, EPT)], srcv)
        pltpu.sync_copy(phys_hbm, physv)
        pltpu.sync_copy(lat_hbm, latv)
        lane = lax.iota(jnp.int32, LANES)
        zi = jnp.zeros((LANES,), jnp.int32)

        def zh(i, carry):
            hist[pl.ds(i * LANES, LANES)] = zi
            return carry

        lax.fori_loop(0, NP // LANES, zh, 0)

        def body(i, carry):
            d = dstv[pl.ds(i * LANES, LANES)]
            sidx = srcv[pl.ds(i * LANES, LANES)]
            sm = lax.rem(sidx, M)
            xd = plsc.load_gather(physv, [d * 2])
            yd = plsc.load_gather(physv, [d * 2 + 1])
            xs = plsc.load_gather(latv, [sm * 2])
            ys = plsc.load_gather(latv, [sm * 2 + 1])
            sl = pl.ds(i * LANES, LANES)
            kinv[0, sl] = xd
            kinv[1, sl] = yd
            kinv[2, sl] = xs
            kinv[3, sl] = ys
            cnts, lmask = plsc.scan_count(d)
            plsc.addupdate_scatter(hist, [d], cnts, mask=lmask)
            return carry

        lax.fori_loop(0, NITER, body, 0)
        pltpu.sync_copy(kinv, kin_hbm.at[:, pl.ds((s * NC + c) * EPTP, EPTP)])
        pltpu.sync_copy(hist, cnt_hbm.at[s * NC + c])

    return kin_kernel


def _make_scatter_kernel(E, NP, F, EPTP):
    EPT = E // NW
    NCHUNK = EPT // CH
    assert NCHUNK % 2 == 1 and CH % 16 == 0
    NG = CH // 16
    RPT = NP // NS
    ZR = 8
    assert RPT % ZR == 0 and RPT % 8 == 0

    @functools.partial(
        pl.kernel, mesh=_mesh(),
        compiler_params=pltpu.CompilerParams(needs_layout_passes=False),
        out_type=jax.ShapeDtypeStruct((NC, NP, C_FEAT), jnp.float32),
        scratch_types=[
            pltpu.VMEM((2, 2, CH), jnp.int32),
            pltpu.VMEM((2, CH, C_FEAT // 2), jnp.int32),
            pltpu.VMEM((2, CH, C_FEAT), jnp.float32),
            pltpu.VMEM((ZR, C_FEAT), jnp.float32),
            pltpu.VMEM_SHARED((NP, C_FEAT), jnp.float32),
            pltpu.SemaphoreType.DMA,
            pltpu.SemaphoreType.DMA,
            pltpu.SemaphoreType.DMA,
            pltpu.SemaphoreType.DMA,
            pltpu.SemaphoreType.DMA,
            pltpu.SemaphoreType.DMA,
            pltpu.SemaphoreType.DMA,
            pltpu.SemaphoreType.DMA,
        ],
    )
    def scat_kernel(dst_hbm, src_hbm, kmat_hbm, f_hbm, part_hbm,
                    idxb, krows, frows, zbuf, accum,
                    ksem0, ksem1, gsem0, gsem1, ssem0, ssem1, isem0, isem1):
        c = lax.axis_index("c")
        s = lax.axis_index("s")
        wid = s * NC + c
        base = wid * EPT
        kbase = wid * EPTP
        zv = jnp.zeros((LANES,), jnp.float32)
        nwords = C_FEAT // LANES

        def zb(i, carry):
            zbuf[i // nwords, pl.ds((i % nwords) * LANES, LANES)] = zv
            return carry

        lax.fori_loop(0, ZR * nwords, zb, 0)
        for r in range(RPT // ZR):
            pltpu.sync_copy(zbuf, accum.at[pl.ds(s * RPT + r * ZR, ZR)])
        plsc.subcore_barrier()

        def issue(a, b, ksem, isem):
            pltpu.async_copy(dst_hbm.at[pl.ds(base + a * CH, CH)],
                             idxb.at[b, 0], isem)
            pltpu.async_copy(src_hbm.at[pl.ds(base + a * CH, CH)],
                             idxb.at[b, 1], isem)
            pltpu.async_copy(kmat_hbm.at[pl.ds(kbase + a * CH, CH)],
                             krows.at[b], ksem)

        def gathers(b, gsem):
            for j in range(NG):
                sv = idxb[b, 1, pl.ds(j * LANES, LANES)]
                pltpu.async_copy(f_hbm.at[sv],
                                 frows.at[b, pl.ds(j * LANES, LANES)], gsem)

        def drain_rows(sem):
            pltpu.make_async_copy(
                f_hbm.at[pl.ds(0, CH)], frows.at[0], sem).wait()

        def drain_half(sem):
            pltpu.make_async_copy(
                kmat_hbm.at[pl.ds(0, CH)], krows.at[0], sem).wait()

        def drain_idx(sem):
            pltpu.make_async_copy(
                dst_hbm.at[pl.ds(0, CH)], idxb.at[0, 0], sem).wait()
            pltpu.make_async_copy(
                dst_hbm.at[pl.ds(0, CH)], idxb.at[0, 0], sem).wait()

        def mul(b):
            def mbody(e, carry):
                for j in range(C_FEAT // 32):
                    kp = krows[b, e, pl.ds(j * LANES, LANES)]
                    kb = plsc.bitcast(kp, jnp.bfloat16)
                    klo, khi = plsc.unpack(kb, format=plsc.PackFormat.INTERLEAVED)
                    clo = pl.ds(j * LANES, LANES)
                    chi = pl.ds(C_FEAT // 2 + j * LANES, LANES)
                    frows[b, e, clo] = klo * frows[b, e, clo]
                    frows[b, e, chi] = khi * frows[b, e, chi]
                return carry
            lax.fori_loop(0, CH, mbody, 0)

        def scatters(b, ssem):
            for j in range(NG):
                dv = idxb[b, 0, pl.ds(j * LANES, LANES)]
                pltpu.async_copy(frows.at[b, pl.ds(j * LANES, LANES)],
                                 accum.at[dv], ssem, add=True)

        issue(0, 0, ksem0, isem0)
        drain_idx(isem0)
        gathers(0, gsem0)

        def pair(p, carry):
            a = 2 * p
            issue(a + 1, 1, ksem1, isem1)
            drain_half(ksem0)
            drain_rows(gsem0)

            @pl.when(p > 0)
            def _():
                drain_rows(ssem1)

            drain_idx(isem1)
            gathers(1, gsem1)
            mul(0)
            scatters(0, ssem0)
            drain_half(ksem1)
            drain_rows(gsem1)
            drain_rows(ssem0)

            @pl.when(a + 2 < NCHUNK)
            def _():
                issue(a + 2, 0, ksem0, isem0)
                drain_idx(isem0)
                gathers(0, gsem0)

            mul(1)
            scatters(1, ssem1)
            return carry

        lax.fori_loop(0, NCHUNK // 2, pair, 0)
        drain_half(ksem0)
        drain_rows(gsem0)
        drain_rows(ssem1)
        mul(0)
        scatters(0, ssem0)
        drain_rows(ssem0)
        plsc.subcore_barrier()
        pltpu.sync_copy(accum.at[pl.ds(s * RPT, RPT)],
                        part_hbm.at[c, pl.ds(s * RPT, RPT)])

    return scat_kernel


def _edge_mlp_body(kin_ref, w0, b0, w1, b1, w2, b2, out_ref):
    x = kin_ref[...]
    h0 = jax.lax.dot_general(x, w0[...], (((0,), (0,)), ((), ())))
    h = jax.nn.gelu((h0 + b0[...]).astype(jnp.bfloat16))
    h1 = jnp.dot(h, w1[...], preferred_element_type=jnp.float32)
    h = jax.nn.gelu((h1 + b1[...]).astype(jnp.bfloat16))
    k = jnp.dot(h, w2[...], preferred_element_type=jnp.float32) + b2[...]
    nh = k.shape[1] // 2
    lo = jax.lax.bitcast_convert_type(
        k[:, :nh].astype(jnp.bfloat16), jnp.int16).astype(jnp.int32)
    hi = jax.lax.bitcast_convert_type(
        k[:, nh:].astype(jnp.bfloat16), jnp.int16).astype(jnp.int32)
    out_ref[...] = (lo & 0xFFFF) | (hi << 16)


def _proj_body(part_ref, cnt_ref, p0, b0, p1, b1, out_ref):
    p = part_ref[0] + part_ref[1]
    cnt = jnp.sum(cnt_ref[...], axis=0).astype(jnp.float32)
    dec = p / jnp.maximum(cnt, 1.0)[:, None]
    h = jax.nn.gelu(jnp.dot(dec, p0[...]) + b0[...])
    out_ref[...] = jnp.dot(h, p1[...]) + b1[...]


def _full(shape):
    return pl.BlockSpec(shape, lambda i: (0,) * len(shape))


def kernel(rndata_batched, phys_pos, latent_tokens, edge_index,
           K0_w, K0_b, K1_w, K1_b, K2_w, K2_b, P0_w, P0_b, P1_w, P1_b):
    B, M, C = rndata_batched.shape
    N = phys_pos.shape[0]
    E = edge_index.shape[1]
    assert E % (NW * CH) == 0 and N % NS == 0

    dst = edge_index[0]
    src = edge_index[1]
    f_flat = rndata_batched.reshape(B * M, C)

    NP = ((N + 128 * NS - 1) // (128 * NS)) * (128 * NS)
    EPT = E // NW
    EPTP = ((EPT + 127) // 128) * 128
    E_pad = NW * EPTP
    kin, cnt = _make_kin_kernel(E, N, M, NP, EPTP)(
        dst, src, phys_pos.reshape(-1), latent_tokens.reshape(-1))

    BE = 4096
    kmat = pl.pallas_call(
        _edge_mlp_body,
        grid=(E_pad // BE,),
        in_specs=[
            pl.BlockSpec((4, BE), lambda i: (0, i)),
            _full((2 * phys_pos.shape[1], 64)), _full((1, 64)),
            _full((64, 64)), _full((1, 64)),
            _full((64, C)), _full((1, C)),
        ],
        out_specs=pl.BlockSpec((BE, C // 2), lambda i: (i, 0)),
        out_shape=jax.ShapeDtypeStruct((E_pad, C // 2), jnp.int32),
    )(kin, K0_w, K0_b.reshape(1, -1),
      K1_w.astype(jnp.bfloat16), K1_b.reshape(1, -1),
      K2_w.astype(jnp.bfloat16), K2_b.reshape(1, -1))

    part = _make_scatter_kernel(E, NP, B * M, EPTP)(dst, src, kmat, f_flat)

    BN = 1024
    out = pl.pallas_call(
        _proj_body,
        grid=(NP // BN,),
        in_specs=[
            pl.BlockSpec((NC, BN, C_FEAT), lambda i: (0, i, 0)),
            pl.BlockSpec((NW, BN), lambda i: (0, i)),
            _full((C, P0_w.shape[1])), _full((1, P0_w.shape[1])),
            _full((P0_w.shape[1], P1_w.shape[1])), _full((1, P1_w.shape[1])),
        ],
        out_specs=pl.BlockSpec((BN, P1_w.shape[1]), lambda i: (i, 0)),
        out_shape=jax.ShapeDtypeStruct((NP, P1_w.shape[1]), jnp.float32),
    )(part, cnt, P0_w, P0_b.reshape(1, -1), P1_w, P1_b.reshape(1, -1))
    return out[:N]


# speedup vs baseline: 1.0446x; 1.0446x over previous
import functools

import jax
import jax.numpy as jnp
from jax import lax
from jax.experimental import pallas as pl
from jax.experimental.pallas import tpu as pltpu
from jax.experimental.pallas import tpu_sc as plsc

NC = 2
NS = 16
NW = NC * NS
LANES = 16

CH = 80
C_FEAT = 128


def _mesh():
    return plsc.VectorSubcoreMesh(
        core_axis_name="c", subcore_axis_name="s",
        num_cores=NC, num_subcores=NS)


def _make_kin_kernel(E, N, M, NP, EPTP):
    EPT = E // NW
    NITER = EPT // LANES

    @functools.partial(
        pl.kernel, mesh=_mesh(),
        compiler_params=pltpu.CompilerParams(needs_layout_passes=False),
        out_type=(jax.ShapeDtypeStruct((4, NW * EPTP), jnp.float32),
                  jax.ShapeDtypeStruct((NW, NP), jnp.int32)),
        scratch_types=[
            pltpu.VMEM((EPT,), jnp.int32),
            pltpu.VMEM((EPT,), jnp.int32),
            pltpu.VMEM((N * 2,), jnp.float32),
            pltpu.VMEM((M * 2,), jnp.float32),
            pltpu.VMEM((4, EPTP), jnp.float32),
            pltpu.VMEM((NP,), jnp.int32),
        ],
    )
    def kin_kernel(dst_hbm, src_hbm, phys_hbm, lat_hbm, kin_hbm, cnt_hbm,
                   dstv, srcv, physv, latv, kinv, hist):
        c = lax.axis_index("c")
        s = lax.axis_index("s")
        base = (s * NC + c) * EPT
        pltpu.sync_copy(dst_hbm.at[pl.ds(base, EPT)], dstv)
        pltpu.sync_copy(src_hbm.at[pl.ds(base, EPT)], srcv)
        pltpu.sync_copy(phys_hbm, physv)
        pltpu.sync_copy(lat_hbm, latv)
        lane = lax.iota(jnp.int32, LANES)
        zi = jnp.zeros((LANES,), jnp.int32)

        def zh(i, carry):
            hist[pl.ds(i * LANES, LANES)] = zi
            return carry

        lax.fori_loop(0, NP // LANES, zh, 0)

        def body(i, carry):
            d = dstv[pl.ds(i * LANES, LANES)]
            sidx = srcv[pl.ds(i * LANES, LANES)]
            sm = lax.rem(sidx, M)
            xd = plsc.load_gather(physv, [d * 2])
            yd = plsc.load_gather(physv, [d * 2 + 1])
            xs = plsc.load_gather(latv, [sm * 2])
            ys = plsc.load_gather(latv, [sm * 2 + 1])
            sl = pl.ds(i * LANES, LANES)
            kinv[0, sl] = xd
            kinv[1, sl] = yd
            kinv[2, sl] = xs
            kinv[3, sl] = ys
            cnts, lmask = plsc.scan_count(d)
            plsc.addupdate_scatter(hist, [d], cnts, mask=lmask)
            return carry

        lax.fori_loop(0, NITER, body, 0)
        pltpu.sync_copy(kinv, kin_hbm.at[:, pl.ds((s * NC + c) * EPTP, EPTP)])
        pltpu.sync_copy(hist, cnt_hbm.at[s * NC + c])

    return kin_kernel


def _make_scatter_kernel(E, NP, F, EPTP):
    EPT = E // NW
    NCHUNK = EPT // CH
    assert NCHUNK % 2 == 1 and CH % 16 == 0
    NG = CH // 16
    RPT = NP // NS
    ZR = 8
    assert RPT % ZR == 0 and RPT % 8 == 0

    @functools.partial(
        pl.kernel, mesh=_mesh(),
        compiler_params=pltpu.CompilerParams(needs_layout_passes=False),
        out_type=jax.ShapeDtypeStruct((NC, NP, C_FEAT), jnp.float32),
        scratch_types=[
            pltpu.VMEM((2, 2, CH), jnp.int32),
            pltpu.VMEM((2, CH, C_FEAT), jnp.float32),
            pltpu.VMEM((2, CH, C_FEAT), jnp.float32),
            pltpu.VMEM((ZR, C_FEAT), jnp.float32),
            pltpu.VMEM_SHARED((NP, C_FEAT), jnp.float32),
            pltpu.SemaphoreType.DMA,
            pltpu.SemaphoreType.DMA,
            pltpu.SemaphoreType.DMA,
            pltpu.SemaphoreType.DMA,
            pltpu.SemaphoreType.DMA,
            pltpu.SemaphoreType.DMA,
            pltpu.SemaphoreType.DMA,
            pltpu.SemaphoreType.DMA,
        ],
    )
    def scat_kernel(dst_hbm, src_hbm, kmat_hbm, f_hbm, part_hbm,
                    idxb, krows, frows, zbuf, accum,
                    ksem0, ksem1, gsem0, gsem1, ssem0, ssem1, isem0, isem1):
        c = lax.axis_index("c")
        s = lax.axis_index("s")
        wid = s * NC + c
        base = wid * EPT
        kbase = wid * EPTP
        zv = jnp.zeros((LANES,), jnp.float32)
        nwords = C_FEAT // LANES

        def zb(i, carry):
            zbuf[i // nwords, pl.ds((i % nwords) * LANES, LANES)] = zv
            return carry

        lax.fori_loop(0, ZR * nwords, zb, 0)
        for r in range(RPT // ZR):
            pltpu.sync_copy(zbuf, accum.at[pl.ds(s * RPT + r * ZR, ZR)])
        plsc.subcore_barrier()

        def issue(a, b, ksem, isem):
            pltpu.async_copy(dst_hbm.at[pl.ds(base + a * CH, CH)],
                             idxb.at[b, 0], isem)
            pltpu.async_copy(src_hbm.at[pl.ds(base + a * CH, CH)],
                             idxb.at[b, 1], isem)
            pltpu.async_copy(kmat_hbm.at[pl.ds(kbase + a * CH, CH)],
                             krows.at[b], ksem)

        def gathers(b, gsem):
            for j in range(NG):
                sv = idxb[b, 1, pl.ds(j * LANES, LANES)]
                pltpu.async_copy(f_hbm.at[sv],
                                 frows.at[b, pl.ds(j * LANES, LANES)], gsem)

        def drain_rows(sem):
            pltpu.make_async_copy(
                f_hbm.at[pl.ds(0, CH)], frows.at[0], sem).wait()

        def drain_half(sem):
            pltpu.make_async_copy(
                kmat_hbm.at[pl.ds(0, CH)], krows.at[0], sem).wait()

        def drain_idx(sem):
            pltpu.make_async_copy(
                dst_hbm.at[pl.ds(0, CH)], idxb.at[0, 0], sem).wait()
            pltpu.make_async_copy(
                dst_hbm.at[pl.ds(0, CH)], idxb.at[0, 0], sem).wait()

        def mul(b):
            def mbody(e, carry):
                for j in range(C_FEAT // LANES):
                    col = pl.ds(j * LANES, LANES)
                    frows[b, e, col] = krows[b, e, col] * frows[b, e, col]
                return carry
            lax.fori_loop(0, CH, mbody, 0)

        def scatters(b, ssem):
            for j in range(NG):
                dv = idxb[b, 0, pl.ds(j * LANES, LANES)]
                pltpu.async_copy(frows.at[b, pl.ds(j * LANES, LANES)],
                                 accum.at[dv], ssem, add=True)

        issue(0, 0, ksem0, isem0)
        drain_idx(isem0)
        gathers(0, gsem0)

        def pair(p, carry):
            a = 2 * p
            issue(a + 1, 1, ksem1, isem1)
            drain_half(ksem0)
            drain_rows(gsem0)

            @pl.when(p > 0)
            def _():
                drain_rows(ssem1)

            drain_idx(isem1)
            gathers(1, gsem1)
            mul(0)
            scatters(0, ssem0)
            drain_half(ksem1)
            drain_rows(gsem1)
            drain_rows(ssem0)

            @pl.when(a + 2 < NCHUNK)
            def _():
                issue(a + 2, 0, ksem0, isem0)
                drain_idx(isem0)
                gathers(0, gsem0)

            mul(1)
            scatters(1, ssem1)
            return carry

        lax.fori_loop(0, NCHUNK // 2, pair, 0)
        drain_half(ksem0)
        drain_rows(gsem0)
        drain_rows(ssem1)
        mul(0)
        scatters(0, ssem0)
        drain_rows(ssem0)
        plsc.subcore_barrier()
        pltpu.sync_copy(accum.at[pl.ds(s * RPT, RPT)],
                        part_hbm.at[c, pl.ds(s * RPT, RPT)])

    return scat_kernel


def _edge_mlp_body(kin_ref, w0, b0, w1, b1, w2, b2, out_ref):
    x = kin_ref[...]
    h0 = jax.lax.dot_general(x, w0[...], (((0,), (0,)), ((), ())))
    h = jax.nn.gelu((h0 + b0[...]).astype(jnp.bfloat16))
    h1 = jnp.dot(h, w1[...], preferred_element_type=jnp.float32)
    h = jax.nn.gelu((h1 + b1[...]).astype(jnp.bfloat16))
    out_ref[...] = jnp.dot(h, w2[...], preferred_element_type=jnp.float32) \
        + b2[...]


def _proj_body(part_ref, cnt_ref, p0, b0, p1, b1, out_ref):
    p = part_ref[0] + part_ref[1]
    cnt = jnp.sum(cnt_ref[...], axis=0).astype(jnp.float32)
    dec = p / jnp.maximum(cnt, 1.0)[:, None]
    h = jax.nn.gelu(jnp.dot(dec, p0[...]) + b0[...])
    out_ref[...] = jnp.dot(h, p1[...]) + b1[...]


def _full(shape):
    return pl.BlockSpec(shape, lambda i: (0,) * len(shape))


def kernel(rndata_batched, phys_pos, latent_tokens, edge_index,
           K0_w, K0_b, K1_w, K1_b, K2_w, K2_b, P0_w, P0_b, P1_w, P1_b):
    B, M, C = rndata_batched.shape
    N = phys_pos.shape[0]
    E = edge_index.shape[1]
    assert E % (NW * CH) == 0 and N % NS == 0

    dst = edge_index[0]
    src = edge_index[1]
    f_flat = rndata_batched.reshape(B * M, C)

    NP = ((N + 128 * NS - 1) // (128 * NS)) * (128 * NS)
    EPT = E // NW
    EPTP = ((EPT + 127) // 128) * 128
    E_pad = NW * EPTP
    kin, cnt = _make_kin_kernel(E, N, M, NP, EPTP)(
        dst, src, phys_pos.reshape(-1), latent_tokens.reshape(-1))

    BE = 4096
    kmat = pl.pallas_call(
        _edge_mlp_body,
        grid=(E_pad // BE,),
        in_specs=[
            pl.BlockSpec((4, BE), lambda i: (0, i)),
            _full((2 * phys_pos.shape[1], 64)), _full((1, 64)),
            _full((64, 64)), _full((1, 64)),
            _full((64, C)), _full((1, C)),
        ],
        out_specs=pl.BlockSpec((BE, C), lambda i: (i, 0)),
        out_shape=jax.ShapeDtypeStruct((E_pad, C), jnp.float32),
    )(kin, K0_w, K0_b.reshape(1, -1),
      K1_w.astype(jnp.bfloat16), K1_b.reshape(1, -1),
      K2_w.astype(jnp.bfloat16), K2_b.reshape(1, -1))

    part = _make_scatter_kernel(E, NP, B * M, EPTP)(dst, src, kmat, f_flat)

    BN = 1024
    out = pl.pallas_call(
        _proj_body,
        grid=(NP // BN,),
        in_specs=[
            pl.BlockSpec((NC, BN, C_FEAT), lambda i: (0, i, 0)),
            pl.BlockSpec((NW, BN), lambda i: (0, i)),
            _full((C, P0_w.shape[1])), _full((1, P0_w.shape[1])),
            _full((P0_w.shape[1], P1_w.shape[1])), _full((1, P1_w.shape[1])),
        ],
        out_specs=pl.BlockSpec((BN, P1_w.shape[1]), lambda i: (i, 0)),
        out_shape=jax.ShapeDtypeStruct((NP, P1_w.shape[1]), jnp.float32),
    )(part, cnt, P0_w, P0_b.reshape(1, -1), P1_w, P1_b.reshape(1, -1))
    return out[:N]


# speedup vs baseline: 1.1007x; 1.0537x over previous
import functools

import jax
import jax.numpy as jnp
from jax import lax
from jax.experimental import pallas as pl
from jax.experimental.pallas import tpu as pltpu
from jax.experimental.pallas import tpu_sc as plsc

NC = 2
NS = 16
NW = NC * NS
LANES = 16

CH = 80
C_FEAT = 128


def _mesh():
    return plsc.VectorSubcoreMesh(
        core_axis_name="c", subcore_axis_name="s",
        num_cores=NC, num_subcores=NS)


def _make_kin_kernel(E, N, M, NP, EPTP):
    EPT = E // NW
    NITER = EPT // LANES

    @functools.partial(
        pl.kernel, mesh=_mesh(),
        compiler_params=pltpu.CompilerParams(needs_layout_passes=False),
        out_type=(jax.ShapeDtypeStruct((4, NW * EPTP), jnp.float32),
                  jax.ShapeDtypeStruct((NW, NP), jnp.int32)),
        scratch_types=[
            pltpu.VMEM((EPT,), jnp.int32),
            pltpu.VMEM((EPT,), jnp.int32),
            pltpu.VMEM((N * 2,), jnp.float32),
            pltpu.VMEM((M * 2,), jnp.float32),
            pltpu.VMEM((4, EPTP), jnp.float32),
            pltpu.VMEM((NP,), jnp.int32),
        ],
    )
    def kin_kernel(dst_hbm, src_hbm, phys_hbm, lat_hbm, kin_hbm, cnt_hbm,
                   dstv, srcv, physv, latv, kinv, hist):
        c = lax.axis_index("c")
        s = lax.axis_index("s")
        base = (s * NC + c) * EPT
        pltpu.sync_copy(dst_hbm.at[pl.ds(base, EPT)], dstv)
        pltpu.sync_copy(src_hbm.at[pl.ds(base, EPT)], srcv)
        pltpu.sync_copy(phys_hbm, physv)
        pltpu.sync_copy(lat_hbm, latv)
        lane = lax.iota(jnp.int32, LANES)
        zi = jnp.zeros((LANES,), jnp.int32)

        def zh(i, carry):
            hist[pl.ds(i * LANES, LANES)] = zi
            return carry

        lax.fori_loop(0, NP // LANES, zh, 0)

        def body(i, carry):
            d = dstv[pl.ds(i * LANES, LANES)]
            sidx = srcv[pl.ds(i * LANES, LANES)]
            sm = lax.rem(sidx, M)
            xd = plsc.load_gather(physv, [d * 2])
            yd = plsc.load_gather(physv, [d * 2 + 1])
            xs = plsc.load_gather(latv, [sm * 2])
            ys = plsc.load_gather(latv, [sm * 2 + 1])
            sl = pl.ds(i * LANES, LANES)
            kinv[0, sl] = xd
            kinv[1, sl] = yd
            kinv[2, sl] = xs
            kinv[3, sl] = ys
            cnts, lmask = plsc.scan_count(d)
            plsc.addupdate_scatter(hist, [d], cnts, mask=lmask)
            return carry

        lax.fori_loop(0, NITER, body, 0)
        pltpu.sync_copy(kinv, kin_hbm.at[:, pl.ds((s * NC + c) * EPTP, EPTP)])
        pltpu.sync_copy(hist, cnt_hbm.at[s * NC + c])

    return kin_kernel


def _make_scatter_kernel(E, NP, F, HEPT, h):
    EPT = E // NW
    NCHUNK_ALL = EPT // CH
    NCH0 = HEPT // CH
    NCHUNK = NCH0 if h == 0 else NCHUNK_ALL - NCH0
    EOFF = h * HEPT
    assert CH % 16 == 0
    NG = CH // 16
    RPT = NP // NS
    ZR = 8
    assert RPT % ZR == 0 and RPT % 8 == 0

    @functools.partial(
        pl.kernel, mesh=_mesh(),
        compiler_params=pltpu.CompilerParams(needs_layout_passes=False),
        out_type=jax.ShapeDtypeStruct((NC, NP, C_FEAT), jnp.float32),
        scratch_types=[
            pltpu.VMEM((2, 2, CH), jnp.int32),
            pltpu.VMEM((2, CH, C_FEAT), jnp.float32),
            pltpu.VMEM((2, CH, C_FEAT), jnp.float32),
            pltpu.VMEM((ZR, C_FEAT), jnp.float32),
            pltpu.VMEM_SHARED((NP, C_FEAT), jnp.float32),
            pltpu.SemaphoreType.DMA,
            pltpu.SemaphoreType.DMA,
            pltpu.SemaphoreType.DMA,
            pltpu.SemaphoreType.DMA,
            pltpu.SemaphoreType.DMA,
            pltpu.SemaphoreType.DMA,
            pltpu.SemaphoreType.DMA,
            pltpu.SemaphoreType.DMA,
        ],
    )
    def scat_kernel(dst_hbm, src_hbm, kmat_hbm, f_hbm, part_hbm,
                    idxb, krows, frows, zbuf, accum,
                    ksem0, ksem1, gsem0, gsem1, ssem0, ssem1, isem0, isem1):
        c = lax.axis_index("c")
        s = lax.axis_index("s")
        wid = s * NC + c
        base = wid * EPT + EOFF
        kbase = wid * HEPT
        zv = jnp.zeros((LANES,), jnp.float32)
        nwords = C_FEAT // LANES

        def zb(i, carry):
            zbuf[i // nwords, pl.ds((i % nwords) * LANES, LANES)] = zv
            return carry

        lax.fori_loop(0, ZR * nwords, zb, 0)
        for r in range(RPT // ZR):
            pltpu.sync_copy(zbuf, accum.at[pl.ds(s * RPT + r * ZR, ZR)])
        plsc.subcore_barrier()

        def issue(a, b, ksem, isem):
            pltpu.async_copy(dst_hbm.at[pl.ds(base + a * CH, CH)],
                             idxb.at[b, 0], isem)
            pltpu.async_copy(src_hbm.at[pl.ds(base + a * CH, CH)],
                             idxb.at[b, 1], isem)
            pltpu.async_copy(kmat_hbm.at[pl.ds(kbase + a * CH, CH)],
                             krows.at[b], ksem)

        def gathers(b, gsem):
            for j in range(NG):
                sv = idxb[b, 1, pl.ds(j * LANES, LANES)]
                pltpu.async_copy(f_hbm.at[sv],
                                 frows.at[b, pl.ds(j * LANES, LANES)], gsem)

        def drain_rows(sem):
            pltpu.make_async_copy(
                f_hbm.at[pl.ds(0, CH)], frows.at[0], sem).wait()

        def drain_half(sem):
            pltpu.make_async_copy(
                kmat_hbm.at[pl.ds(0, CH)], krows.at[0], sem).wait()

        def drain_idx(sem):
            pltpu.make_async_copy(
                dst_hbm.at[pl.ds(0, CH)], idxb.at[0, 0], sem).wait()
            pltpu.make_async_copy(
                dst_hbm.at[pl.ds(0, CH)], idxb.at[0, 0], sem).wait()

        def mul(b):
            def mbody(e, carry):
                for j in range(C_FEAT // LANES):
                    col = pl.ds(j * LANES, LANES)
                    frows[b, e, col] = krows[b, e, col] * frows[b, e, col]
                return carry
            lax.fori_loop(0, CH, mbody, 0)

        def scatters(b, ssem):
            for j in range(NG):
                dv = idxb[b, 0, pl.ds(j * LANES, LANES)]
                pltpu.async_copy(frows.at[b, pl.ds(j * LANES, LANES)],
                                 accum.at[dv], ssem, add=True)

        issue(0, 0, ksem0, isem0)
        drain_idx(isem0)
        gathers(0, gsem0)

        def pair(p, carry):
            a = 2 * p
            issue(a + 1, 1, ksem1, isem1)
            drain_half(ksem0)
            drain_rows(gsem0)

            @pl.when(p > 0)
            def _():
                drain_rows(ssem1)

            drain_idx(isem1)
            gathers(1, gsem1)
            mul(0)
            scatters(0, ssem0)
            drain_half(ksem1)
            drain_rows(gsem1)
            drain_rows(ssem0)

            @pl.when(a + 2 < NCHUNK)
            def _():
                issue(a + 2, 0, ksem0, isem0)
                drain_idx(isem0)
                gathers(0, gsem0)

            mul(1)
            scatters(1, ssem1)
            return carry

        lax.fori_loop(0, NCHUNK // 2, pair, 0)
        if NCHUNK % 2 == 1:
            drain_half(ksem0)
            drain_rows(gsem0)
            drain_rows(ssem1)
            mul(0)
            scatters(0, ssem0)
            drain_rows(ssem0)
        else:
            drain_rows(ssem1)
        plsc.subcore_barrier()
        pltpu.sync_copy(accum.at[pl.ds(s * RPT, RPT)],
                        part_hbm.at[c, pl.ds(s * RPT, RPT)])

    return scat_kernel


def _edge_mlp_body(kin_ref, w0, b0, w1, b1, w2, b2, out_ref):
    x = kin_ref[...]
    h0 = jax.lax.dot_general(x, w0[...], (((0,), (0,)), ((), ())))
    h = jax.nn.gelu((h0 + b0[...]).astype(jnp.bfloat16))
    h1 = jnp.dot(h, w1[...], preferred_element_type=jnp.float32)
    h = jax.nn.gelu((h1 + b1[...]).astype(jnp.bfloat16))
    out_ref[...] = jnp.dot(h, w2[...], preferred_element_type=jnp.float32) \
        + b2[...]


def _proj_body(part_ref, part1_ref, cnt_ref, p0, b0, p1, b1, out_ref):
    p = (part_ref[0] + part_ref[1]) + (part1_ref[0] + part1_ref[1])
    cnt = jnp.sum(cnt_ref[...], axis=0).astype(jnp.float32)
    dec = p / jnp.maximum(cnt, 1.0)[:, None]
    h = jax.nn.gelu(jnp.dot(dec, p0[...]) + b0[...])
    out_ref[...] = jnp.dot(h, p1[...]) + b1[...]


def _full(shape):
    return pl.BlockSpec(shape, lambda i: (0,) * len(shape))


def kernel(rndata_batched, phys_pos, latent_tokens, edge_index,
           K0_w, K0_b, K1_w, K1_b, K2_w, K2_b, P0_w, P0_b, P1_w, P1_b):
    B, M, C = rndata_batched.shape
    N = phys_pos.shape[0]
    E = edge_index.shape[1]
    assert E % (NW * CH) == 0 and N % NS == 0

    dst = edge_index[0]
    src = edge_index[1]
    f_flat = rndata_batched.reshape(B * M, C)

    NP = ((N + 128 * NS - 1) // (128 * NS)) * (128 * NS)
    EPT = E // NW
    EPTP = ((EPT + 1279) // 1280) * 1280
    E_pad = NW * EPTP
    kin, cnt = _make_kin_kernel(E, N, M, NP, EPTP)(
        dst, src, phys_pos.reshape(-1), latent_tokens.reshape(-1))

    HEPT = EPTP // 2
    mlp_in = [
        pl.BlockSpec((4, HEPT), None),
        _full((2 * phys_pos.shape[1], 64)), _full((1, 64)),
        _full((64, 64)), _full((1, 64)),
        _full((64, C)), _full((1, C)),
    ]
    wargs = (K0_w, K0_b.reshape(1, -1),
             K1_w.astype(jnp.bfloat16), K1_b.reshape(1, -1),
             K2_w.astype(jnp.bfloat16), K2_b.reshape(1, -1))

    def mlp_half(h):
        specs = list(mlp_in)
        specs[0] = pl.BlockSpec((4, HEPT), lambda w: (0, 2 * w + h))
        return pl.pallas_call(
            _edge_mlp_body,
            grid=(NW,),
            in_specs=specs,
            out_specs=pl.BlockSpec((HEPT, C), lambda w: (w, 0)),
            out_shape=jax.ShapeDtypeStruct((NW * HEPT, C), jnp.float32),
        )(kin, *wargs)

    kmat0 = mlp_half(0)
    kmat1 = mlp_half(1)
    part0 = _make_scatter_kernel(E, NP, B * M, HEPT, 0)(dst, src, kmat0, f_flat)
    part1 = _make_scatter_kernel(E, NP, B * M, HEPT, 1)(dst, src, kmat1, f_flat)

    BN = 1024
    out = pl.pallas_call(
        _proj_body,
        grid=(NP // BN,),
        in_specs=[
            pl.BlockSpec((NC, BN, C_FEAT), lambda i: (0, i, 0)),
            pl.BlockSpec((NC, BN, C_FEAT), lambda i: (0, i, 0)),
            pl.BlockSpec((NW, BN), lambda i: (0, i)),
            _full((C, P0_w.shape[1])), _full((1, P0_w.shape[1])),
            _full((P0_w.shape[1], P1_w.shape[1])), _full((1, P1_w.shape[1])),
        ],
        out_specs=pl.BlockSpec((BN, P1_w.shape[1]), lambda i: (i, 0)),
        out_shape=jax.ShapeDtypeStruct((NP, P1_w.shape[1]), jnp.float32),
    )(part0, part1, cnt, P0_w, P0_b.reshape(1, -1), P1_w, P1_b.reshape(1, -1))
    return out[:N]
